# Initial kernel scaffold; baseline (speedup 1.0000x reference)
import jax
import jax.numpy as jnp
from jax.experimental import pallas as pl


def kernel(x, edge_index, edge_attr, params):
    raise NotImplementedError("write your pallas kernel here")


# speedup vs baseline: 1.7100x; 1.7100x over previous
import functools

import jax
import jax.numpy as jnp
from jax import lax
from jax.experimental import pallas as pl
from jax.experimental.pallas import tpu as pltpu
from jax.experimental.pallas import tpu_sc as plsc

LEAK = 0.1
EPS = 1e-5
NC = 2
NS = 16
NW = NC * NS
CH = 128

_SC_PARAMS = pltpu.CompilerParams(use_tc_tiling_on_sc=False)


def _mesh():
    return plsc.VectorSubcoreMesh(core_axis_name="c", subcore_axis_name="s")


def _chunks(total, step):
    out, off = [], 0
    while off < total:
        sz = min(step, total - off)
        out.append((off, sz))
        off += sz
    return out


def _round8(v):
    return (v + 7) // 8 * 8


def _fill(ref, rows, width, value):
    v = jnp.full((16,), value, jnp.float32)

    def fb(i, carry):
        for w0 in range(width // 16):
            ref[i, pl.ds(w0 * 16, 16)] = v
        return carry

    lax.fori_loop(0, rows, fb, 0)


@functools.lru_cache(None)
def _degrees_kernel(n, e):
    q = e // NS
    f = q // CH
    rem = q - f * CH
    nb = 2
    rnds = f // nb
    stripe = _round8(-(-n // NS))
    n_pad = stripe * NS
    dw = 16

    scratch = (
        [pltpu.VMEM((CH,), jnp.int32)] * nb
        + [pltpu.VMEM((rem,), jnp.int32)]
        + [pltpu.VMEM((CH, dw), jnp.float32)]
        + [pltpu.VMEM((1024, dw), jnp.float32)]
        + [pltpu.VMEM_SHARED((n_pad, dw), jnp.float32)]
        + [pltpu.SemaphoreType.DMA] * nb
    )

    def body(eidx, out, *scr):
        idx_v = scr[0:nb]
        idx_r = scr[nb]
        ones_v = scr[nb + 1]
        zb = scr[nb + 2]
        acc = scr[nb + 3]
        si = scr[nb + 4 : nb + 4 + nb]
        c = lax.axis_index("c")
        s = lax.axis_index("s")
        tb = s * q
        r0 = s * stripe
        ebase = c * e + tb

        _fill(zb, 1024, dw, 0.0)
        for off, sz in _chunks(stripe, 1024):
            pltpu.sync_copy(zb.at[pl.ds(0, sz)], acc.at[pl.ds(r0 + off, sz)])
        _fill(ones_v, CH, dw, 1.0)
        plsc.subcore_barrier()

        def fetch(j, b):
            pltpu.async_copy(eidx.at[pl.ds(ebase + j * CH, CH)], idx_v[b], si[b])

        def wait_idx(b):
            pltpu.make_async_copy(eidx.at[pl.ds(0, CH)], idx_v[b], si[b]).wait()

        for b in range(nb):
            fetch(b, b)

        def rbody(r, carry):
            j0 = r * nb
            for b in range(nb):
                j = j0 + b
                wait_idx(b)
                pltpu.sync_copy(ones_v, acc.at[idx_v[b]], add=True)

                @pl.when(j + nb < f)
                def _():
                    fetch(j + nb, b)

            return carry

        lax.fori_loop(0, rnds, rbody, 0)
        if rem:
            pltpu.sync_copy(eidx.at[pl.ds(ebase + f * CH, rem)], idx_r)
            pltpu.sync_copy(ones_v.at[pl.ds(0, rem)], acc.at[idx_r], add=True)
        plsc.subcore_barrier()

        for off, sz in _chunks(stripe, 1024):
            pltpu.sync_copy(acc.at[pl.ds(r0 + off, sz)], zb.at[pl.ds(0, sz)])
            pltpu.sync_copy(zb.at[pl.ds(0, sz)], out.at[c, pl.ds(r0 + off, sz)])

    return pl.kernel(
        body,
        out_type=jax.ShapeDtypeStruct((2, n_pad, dw), jnp.float32),
        mesh=_mesh(),
        scratch_types=scratch,
        compiler_params=_SC_PARAMS,
    )


def _degrees(edge_index, n):
    e = edge_index.shape[1]
    out = _degrees_kernel(n, e)(edge_index.reshape(2 * e))
    return out[0, :n, 0:1], out[1, :n, 0:1]


@functools.lru_cache(None)
def _gather_kernel(n, d, e):
    q = e // NW
    f = q // CH
    rem = q - f * CH
    nb = 5
    assert f % nb == 0
    rnds = f // nb

    scratch = (
        [pltpu.VMEM((CH,), jnp.int32)] * (2 * nb)
        + [pltpu.VMEM((CH, d), jnp.float32)] * (2 * nb)
        + [pltpu.VMEM((rem,), jnp.int32), pltpu.VMEM((rem, d), jnp.float32)]
        + [pltpu.SemaphoreType.DMA] * (4 * nb)
    )

    def body(table, ridx, cidx, out, *scr):
        ir_v = scr[0:nb]
        ic_v = scr[nb : 2 * nb]
        ra_v = scr[2 * nb : 3 * nb]
        rb_v = scr[3 * nb : 4 * nb]
        idx_r = scr[4 * nb]
        rows_r = scr[4 * nb + 1]
        sir = scr[4 * nb + 2 : 4 * nb + 2 + nb]
        sic = scr[4 * nb + 2 + nb : 4 * nb + 2 + 2 * nb]
        soa = scr[4 * nb + 2 + 2 * nb : 4 * nb + 2 + 3 * nb]
        sob = scr[4 * nb + 2 + 3 * nb : 4 * nb + 2 + 4 * nb]
        c = lax.axis_index("c")
        s = lax.axis_index("s")
        base = (s * NC + c) * q

        def fetch(j, b):
            pltpu.async_copy(ridx.at[pl.ds(base + j * CH, CH)], ir_v[b], sir[b])
            pltpu.async_copy(cidx.at[pl.ds(base + j * CH, CH)], ic_v[b], sic[b])

        def wait_idx(b):
            pltpu.make_async_copy(ridx.at[pl.ds(0, CH)], ir_v[b], sir[b]).wait()
            pltpu.make_async_copy(cidx.at[pl.ds(0, CH)], ic_v[b], sic[b]).wait()

        def wait_store(b):
            pltpu.make_async_copy(ra_v[b], out.at[pl.ds(0, CH), pl.ds(0, d)], soa[b]).wait()
            pltpu.make_async_copy(rb_v[b], out.at[pl.ds(0, CH), pl.ds(0, d)], sob[b]).wait()

        def work(j, b):
            pltpu.sync_copy(table.at[ir_v[b]], ra_v[b])
            pltpu.sync_copy(table.at[ic_v[b]], rb_v[b])
            rows = pl.ds(base + j * CH, CH)
            pltpu.async_copy(ra_v[b], out.at[rows, pl.ds(0, d)], soa[b])
            pltpu.async_copy(rb_v[b], out.at[rows, pl.ds(d, d)], sob[b])

        for b in range(nb):
            fetch(b, b)
        for b in range(nb):
            wait_idx(b)
            work(b, b)
            fetch(b + nb, b)

        def rbody(r, carry):
            j0 = r * nb
            for b in range(nb):
                j = j0 + b
                wait_idx(b)
                wait_store(b)
                work(j, b)

                @pl.when(j + nb < f)
                def _():
                    fetch(j + nb, b)

            return carry

        lax.fori_loop(1, rnds, rbody, 0)
        for b in range(nb):
            wait_store(b)
        if rem:
            rows = pl.ds(base + f * CH, rem)
            pltpu.sync_copy(ridx.at[rows], idx_r)
            pltpu.sync_copy(table.at[idx_r], rows_r)
            pltpu.sync_copy(rows_r, out.at[rows, pl.ds(0, d)])
            pltpu.sync_copy(cidx.at[rows], idx_r)
            pltpu.sync_copy(table.at[idx_r], rows_r)
            pltpu.sync_copy(rows_r, out.at[rows, pl.ds(d, d)])

    return pl.kernel(
        body,
        out_type=jax.ShapeDtypeStruct((e, 128), jnp.float32),
        mesh=_mesh(),
        scratch_types=scratch,
        compiler_params=_SC_PARAMS,
    )


def _gather_pair(table, row, col):
    n, d = table.shape
    return _gather_kernel(n, d, row.shape[0])(table, row, col)


@functools.lru_cache(None)
def _scatter_kernel(n, e, dc):
    q = e // NS
    f = q // CH
    rem = q - f * CH
    nb = 2
    rnds = f // nb
    stripe = _round8(-(-n // NS))
    n_pad = stripe * NS

    scratch = (
        [pltpu.VMEM((CH,), jnp.int32)] * nb
        + [pltpu.VMEM((CH, dc), jnp.float32)] * nb
        + [pltpu.VMEM((rem,), jnp.int32), pltpu.VMEM((rem, dc), jnp.float32)]
        + [pltpu.VMEM((128, dc), jnp.float32)]
        + [pltpu.VMEM_SHARED((n_pad, dc), jnp.float32)]
        + [pltpu.SemaphoreType.DMA] * (2 * nb)
    )

    def body(m, colv, out, *scr):
        idx_v = scr[0:nb]
        val_v = scr[nb : 2 * nb]
        idx_r = scr[2 * nb]
        val_r = scr[2 * nb + 1]
        zb = scr[2 * nb + 2]
        acc = scr[2 * nb + 3]
        si = scr[2 * nb + 4 : 2 * nb + 4 + nb]
        sv = scr[2 * nb + 4 + nb : 2 * nb + 4 + 2 * nb]
        c = lax.axis_index("c")
        s = lax.axis_index("s")
        tb = s * q
        r0 = s * stripe
        ccol = c * dc

        _fill(zb, 128, dc, 0.0)
        for off, sz in _chunks(stripe, 128):
            pltpu.sync_copy(zb.at[pl.ds(0, sz)], acc.at[pl.ds(r0 + off, sz)])
        plsc.subcore_barrier()

        def fetch(j, b):
            rows = pl.ds(tb + j * CH, CH)
            pltpu.async_copy(colv.at[rows], idx_v[b], si[b])
            pltpu.async_copy(m.at[rows, pl.ds(ccol, dc)], val_v[b], sv[b])

        for b in range(nb):
            fetch(b, b)

        def rbody(r, carry):
            j0 = r * nb
            for b in range(nb):
                j = j0 + b
                pltpu.make_async_copy(colv.at[pl.ds(0, CH)], idx_v[b], si[b]).wait()
                pltpu.make_async_copy(
                    m.at[pl.ds(0, CH), pl.ds(0, dc)], val_v[b], sv[b]
                ).wait()
                pltpu.sync_copy(val_v[b], acc.at[idx_v[b]], add=True)

                @pl.when(j + nb < f)
                def _():
                    fetch(j + nb, b)

            return carry

        lax.fori_loop(0, rnds, rbody, 0)
        if rem:
            rows = pl.ds(tb + f * CH, rem)
            pltpu.sync_copy(colv.at[rows], idx_r)
            pltpu.sync_copy(m.at[rows, pl.ds(ccol, dc)], val_r)
            pltpu.sync_copy(val_r, acc.at[idx_r], add=True)
        plsc.subcore_barrier()

        for off, sz in _chunks(stripe, 128):
            pltpu.sync_copy(acc.at[pl.ds(r0 + off, sz)], zb.at[pl.ds(0, sz)])
            pltpu.sync_copy(zb.at[pl.ds(0, sz)], out.at[c, pl.ds(r0 + off, sz)])

    return pl.kernel(
        body,
        out_type=jax.ShapeDtypeStruct((NC, n_pad, dc), jnp.float32),
        mesh=_mesh(),
        scratch_types=scratch,
        compiler_params=_SC_PARAMS,
    )


def _scatter_sum(m128, col, n, d):
    e = m128.shape[0]
    dc = d // NC
    return _scatter_kernel(n, e, dc)(m128, col)


def _tc_pass(ins, ws, bias, *, leaky, wvecs=(), head=None, pad_out=False, tile=2000):
    rrows = ins[0][0].shape[0]
    grid = rrows // tile
    assert grid * tile == rrows
    ni = len(ins)
    nw = len(wvecs)
    dout = ws.shape[1]
    srows = 2 + 2 * nw

    def body(*refs):
        in_refs = refs[:ni]
        w_refs = refs[ni : ni + 1]
        sc_refs = refs[ni + 1 : 2 * ni + 1]
        sh_refs = refs[2 * ni + 1 : 3 * ni + 1]
        g_refs = refs[3 * ni + 1 : 4 * ni + 1]
        bt_refs = refs[4 * ni + 1 : 5 * ni + 1]
        b_ref = refs[5 * ni + 1]
        pos = 5 * ni + 2
        wv_refs = refs[pos : pos + nw]
        pos += nw
        if head is not None:
            hw_ref, hb_ref = refs[pos : pos + 2]
            pos += 2
        out_ref = refs[pos]
        acc_ref = refs[pos + 1]
        if head is not None:
            hout_ref = refs[pos + 2]

        parts = []
        for k in range(ni):
            a = in_refs[k][...]
            use = ins[k][2]
            if use != a.shape[1]:
                a = a[:, :use]
            m_k = sc_refs[k][...]
            sq_k = sh_refs[k][...]
            g_k = g_refs[k][...]
            beta_k = bt_refs[k][...]
            parts.append((a - m_k) / sq_k * g_k + beta_k)
        hcat = parts[0] if ni == 1 else jnp.concatenate(parts, axis=1)
        h = jnp.dot(hcat, w_refs[0][...], preferred_element_type=jnp.float32)
        h = h + b_ref[...]
        if leaky:
            h = jnp.where(h >= 0, h, LEAK * h)
        if pad_out:
            out_ref[...] = jnp.concatenate(
                [h, jnp.zeros((h.shape[0], 128 - dout), h.dtype)], axis=1
            )
        else:
            out_ref[...] = h

        rows = [jnp.sum(h, axis=0), jnp.sum(h * h, axis=0)]
        for wv in wv_refs:
            w = wv[...]
            rows.append(jnp.sum(w * h, axis=0))
            rows.append(jnp.sum(w * (h * h), axis=0))
        acc_ref[...] = jnp.concatenate([r[None, :] for r in rows], axis=0)[None]

        if head is not None:
            hout_ref[...] = (
                jnp.dot(h, hw_ref[...], preferred_element_type=jnp.float32)
                + hb_ref[...]
            )

    in_specs = [pl.BlockSpec((tile, bc), lambda i: (i, 0)) for (_, bc, _, _) in ins]
    in_specs += [pl.BlockSpec(ws.shape, lambda i: (0, 0))]
    for _ in range(4):
        in_specs += [pl.BlockSpec((1, u), lambda i: (0, 0)) for (_, _, u, _) in ins]
    in_specs += [pl.BlockSpec((1, dout), lambda i: (0, 0))]
    in_specs += [pl.BlockSpec((tile, 1), lambda i: (i, 0)) for _ in wvecs]
    args = (
        [a for (a, _, _, _) in ins]
        + [ws]
        + [p[0] for (_, _, _, p) in ins]
        + [p[1] for (_, _, _, p) in ins]
        + [p[2] for (_, _, _, p) in ins]
        + [p[3] for (_, _, _, p) in ins]
        + [bias]
        + list(wvecs)
    )
    ocols = 128 if pad_out else dout
    out_shape = [jax.ShapeDtypeStruct((rrows, ocols), jnp.float32)]
    out_specs = [pl.BlockSpec((tile, ocols), lambda i: (i, 0))]
    out_shape.append(jax.ShapeDtypeStruct((grid, srows, dout), jnp.float32))
    out_specs.append(pl.BlockSpec((1, srows, dout), lambda i: (i, 0, 0)))
    if head is not None:
        hw, hb = head
        in_specs += [
            pl.BlockSpec(hw.shape, lambda i: (0, 0)),
            pl.BlockSpec((1, hw.shape[1]), lambda i: (0, 0)),
        ]
        args += [hw, hb]
        out_shape.append(jax.ShapeDtypeStruct((rrows, hw.shape[1]), jnp.float32))
        out_specs.append(pl.BlockSpec((tile, hw.shape[1]), lambda i: (i, 0)))

    res = pl.pallas_call(
        body,
        grid=(grid,),
        in_specs=in_specs,
        out_specs=out_specs,
        out_shape=out_shape,
    )(*args)
    res = list(res)
    res[1] = jnp.sum(res[1], axis=0)
    return res


def _tc_stats(arr, wvecs=(), tile=2000):
    rrows, d = arr.shape
    grid = rrows // tile
    assert grid * tile == rrows
    nw = len(wvecs)
    srows = 2 + 2 * nw

    def body(*refs):
        a_ref = refs[0]
        wv_refs = refs[1 : 1 + nw]
        acc_ref = refs[1 + nw]
        h = a_ref[...]
        rows = [jnp.sum(h, axis=0), jnp.sum(h * h, axis=0)]
        for wv in wv_refs:
            w = wv[...]
            rows.append(jnp.sum(w * h, axis=0))
            rows.append(jnp.sum(w * (h * h), axis=0))
        acc_ref[...] = jnp.concatenate([r[None, :] for r in rows], axis=0)[None]

    in_specs = [pl.BlockSpec((tile, d), lambda i: (i, 0))]
    in_specs += [pl.BlockSpec((tile, 1), lambda i: (i, 0)) for _ in wvecs]
    res = pl.pallas_call(
        body,
        grid=(grid,),
        in_specs=in_specs,
        out_specs=pl.BlockSpec((1, srows, d), lambda i: (i, 0, 0)),
        out_shape=jax.ShapeDtypeStruct((grid, srows, d), jnp.float32),
    )(arr, *wvecs)
    return jnp.sum(res, axis=0)


def _tc_segmean(ssum, cnt, tile=2000):
    rrows = cnt.shape[0]
    dc = ssum.shape[2]
    d = ssum.shape[0] * dc
    grid = rrows // tile

    def body(s_ref, c_ref, out_ref, acc_ref):
        h = jnp.concatenate([s_ref[k] for k in range(NC)], axis=-1)
        h = h / jnp.maximum(c_ref[...], 1.0)
        out_ref[...] = h
        acc_ref[...] = jnp.concatenate(
            [jnp.sum(h, axis=0)[None, :], jnp.sum(h * h, axis=0)[None, :]], axis=0
        )[None]

    out, parts = pl.pallas_call(
        body,
        grid=(grid,),
        in_specs=[
            pl.BlockSpec((NC, tile, dc), lambda i: (0, i, 0)),
            pl.BlockSpec((tile, 1), lambda i: (i, 0)),
        ],
        out_specs=[
            pl.BlockSpec((tile, d), lambda i: (i, 0)),
            pl.BlockSpec((1, 2, d), lambda i: (i, 0, 0)),
        ],
        out_shape=[
            jax.ShapeDtypeStruct((rrows, d), jnp.float32),
            jax.ShapeDtypeStruct((grid, 2, d), jnp.float32),
        ],
    )(ssum, cnt)
    return out, jnp.sum(parts, axis=0)


def _mv(acc0, acc1, denom):
    m = acc0 / denom
    return m, acc1 / denom - m * m


def _bnp(blk, mean, var, lo, hi):
    return (
        mean[None, :],
        jnp.sqrt(var + EPS)[None, :],
        blk["g"][lo:hi][None, :],
        blk["beta"][lo:hi][None, :],
    )


def kernel(x, edge_index, edge_attr, params):
    n = x.shape[0]
    e = edge_attr.shape[0]
    row = edge_index[0]
    col = edge_index[1]

    deg_out, deg_in = _degrees(edge_index, n)

    accx = _tc_stats(x, (deg_out, deg_in))
    mx, vx = _mv(accx[0], accx[1], n)
    mr, vr = _mv(accx[2], accx[3], e)
    mc, vc = _mv(accx[4], accx[5], e)
    accea = _tc_stats(edge_attr)
    me, ve = _mv(accea[0], accea[1], e)

    ea = edge_attr
    for li, lp in enumerate(params["mp"]):
        dn = x.shape[1]
        de = ea.shape[1]
        last = li == len(params["mp"]) - 1

        gx = _gather_pair(x, row, col)

        blk = lp["edge"][0]
        p_g = _bnp(blk, jnp.concatenate([mr, mc]), jnp.concatenate([vr, vc]), 0, 2 * dn)
        p_e = _bnp(blk, me, ve, 2 * dn, 2 * dn + de)
        h, acc = _tc_pass(
            ((gx, 128, 2 * dn, p_g), (ea, de, de, p_e)),
            blk["W"].T,
            blk["b"][None, :],
            leaky=True,
        )
        for bi in (1, 2):
            blk = lp["edge"][bi]
            m1, v1 = _mv(acc[0], acc[1], e)
            p1 = _bnp(blk, m1, v1, 0, 64)
            if bi == 2 and last:
                ep = params["edge_pred"]
                ea, acc, edge_pred = _tc_pass(
                    ((h, 64, 64, p1),), blk["W"].T, blk["b"][None, :],
                    leaky=False, head=(ep["W"].T, ep["b"][None, :]),
                )
            else:
                h, acc = _tc_pass(
                    ((h, 64, 64, p1),), blk["W"].T, blk["b"][None, :],
                    leaky=bi < 2,
                )
                if bi == 2:
                    ea = h
        me, ve = _mv(acc[0], acc[1], e)

        blk = lp["mlp1"][0]
        p_r = _bnp(blk, mr, vr, 0, dn)
        p_e = _bnp(blk, me, ve, dn, dn + 64)
        h, acc = _tc_pass(
            ((gx, 128, dn, p_r), (ea, 64, 64, p_e)),
            blk["W"].T,
            blk["b"][None, :],
            leaky=True,
        )
        blk = lp["mlp1"][1]
        m1, v1 = _mv(acc[0], acc[1], e)
        h, acc = _tc_pass(
            ((h, 64, 64, _bnp(blk, m1, v1, 0, 64)),), blk["W"].T, blk["b"][None, :],
            leaky=True,
        )
        blk = lp["mlp1"][2]
        m1, v1 = _mv(acc[0], acc[1], e)
        msg, _ = _tc_pass(
            ((h, 64, 64, _bnp(blk, m1, v1, 0, 64)),), blk["W"].T, blk["b"][None, :],
            leaky=False, pad_out=True,
        )

        ssum = _scatter_sum(msg, col, n, 64)
        agg, acca = _tc_segmean(ssum, deg_in)
        ma, va = _mv(acca[0], acca[1], n)

        blk = lp["mlp2"][0]
        p_x = _bnp(blk, mx, vx, 0, dn)
        p_a = _bnp(blk, ma, va, dn, dn + 64)
        h, acc = _tc_pass(
            ((x, dn, dn, p_x), (agg, 64, 64, p_a)),
            blk["W"].T,
            blk["b"][None, :],
            leaky=True,
        )
        blk = lp["mlp2"][1]
        m1, v1 = _mv(acc[0], acc[1], n)
        h, acc = _tc_pass(
            ((h, 64, 64, _bnp(blk, m1, v1, 0, 64)),), blk["W"].T, blk["b"][None, :],
            leaky=True,
        )
        blk = lp["mlp2"][2]
        m1, v1 = _mv(acc[0], acc[1], n)
        p1 = _bnp(blk, m1, v1, 0, 64)
        if last:
            npd = params["node_pred"]
            x, acc, node_pred = _tc_pass(
                ((h, 64, 64, p1),), blk["W"].T, blk["b"][None, :],
                leaky=False, head=(npd["W"].T, npd["b"][None, :]),
            )
        else:
            x, acc = _tc_pass(
                ((h, 64, 64, p1),), blk["W"].T, blk["b"][None, :],
                leaky=False, wvecs=(deg_out, deg_in),
            )
            mx, vx = _mv(acc[0], acc[1], n)
            mr, vr = _mv(acc[2], acc[3], e)
            mc, vc = _mv(acc[4], acc[5], e)

    return node_pred, edge_pred
